# Initial kernel scaffold; baseline (speedup 1.0000x reference)
import jax
import jax.numpy as jnp
from jax.experimental import pallas as pl


def kernel(x, edge_index, edge_attr, batch, Wq, bq, Wk, bk, Wv, bv, We, Wskip, bskip, W1, b1, W2, b2):
    raise NotImplementedError("write your pallas kernel here")


# speedup vs baseline: 15.8511x; 15.8511x over previous
import functools

import jax
import jax.numpy as jnp
import numpy as np
from jax import lax
from jax.experimental import pallas as pl
from jax.experimental.pallas import tpu as pltpu
from jax.experimental.pallas import tpu_sc as plsc

N = 10000
E = 320000
D = 128
DE = 16
H = 16
C = 8

NC = 2
NS = 16
NW = NC * NS
EPW = E // NW
B = 40
CHUNKS = EPW // B
RPT = N // NS
AW = D + H

_INV_SQRT_C = 1.0 / float(np.sqrt(C))


def _leaky(v):
    return jnp.maximum(v, 0.01 * v)


def _tc_nodes_pre(x, Wq, bq, Wk, bk, Wv, bv, Wskip, bskip):
    NB = 2000

    def body(x_ref, wq, bq_, wk, bk_, wv, bv_, ws, bs_, q_ref, kv_ref, s_ref):
        xb = x_ref[...]
        q_ref[...] = jnp.dot(xb, wq[...], preferred_element_type=jnp.float32) + bq_[...]
        kv_ref[:, :D] = jnp.dot(xb, wk[...], preferred_element_type=jnp.float32) + bk_[...]
        kv_ref[:, D:] = jnp.dot(xb, wv[...], preferred_element_type=jnp.float32) + bv_[...]
        s_ref[...] = jnp.dot(xb, ws[...], preferred_element_type=jnp.float32) + bs_[...]

    wspec = pl.BlockSpec((D, D), lambda i: (0, 0))
    bspec = pl.BlockSpec((D,), lambda i: (0,))
    return pl.pallas_call(
        body,
        grid=(N // NB,),
        in_specs=[
            pl.BlockSpec((NB, D), lambda i: (i, 0)),
            wspec, bspec, wspec, bspec, wspec, bspec, wspec, bspec,
        ],
        out_specs=[
            pl.BlockSpec((NB, D), lambda i: (i, 0)),
            pl.BlockSpec((NB, 2 * D), lambda i: (i, 0)),
            pl.BlockSpec((NB, D), lambda i: (i, 0)),
        ],
        out_shape=[
            jax.ShapeDtypeStruct((N, D), jnp.float32),
            jax.ShapeDtypeStruct((N, 2 * D), jnp.float32),
            jax.ShapeDtypeStruct((N, D), jnp.float32),
        ],
    )(x, Wq, bq, Wk, bk, Wv, bv, Wskip, bskip)


def _tc_edges_pre(edge_attr, We, W1c, b1):
    EB = 4000

    def body(ea_ref, we, w1c, b1_, eaw_ref, ec_ref):
        eb = ea_ref[...]
        eaw_ref[...] = jnp.dot(eb, we[...], preferred_element_type=jnp.float32)
        ec_ref[...] = jnp.dot(eb, w1c[...], preferred_element_type=jnp.float32) + b1_[...]

    return pl.pallas_call(
        body,
        grid=(E // EB,),
        in_specs=[
            pl.BlockSpec((EB, DE), lambda i: (i, 0)),
            pl.BlockSpec((DE, D), lambda i: (0, 0)),
            pl.BlockSpec((DE, DE), lambda i: (0, 0)),
            pl.BlockSpec((DE,), lambda i: (0,)),
        ],
        out_specs=[
            pl.BlockSpec((EB, D), lambda i: (i, 0)),
            pl.BlockSpec((EB, DE), lambda i: (i, 0)),
        ],
        out_shape=[
            jax.ShapeDtypeStruct((E, D), jnp.float32),
            jax.ShapeDtypeStruct((E, DE), jnp.float32),
        ],
    )(edge_attr, We, W1c, b1)


def _sc_attention(src, dst, q, kv, ea):
    mesh = plsc.VectorSubcoreMesh(core_axis_name="c", subcore_axis_name="s")

    @functools.partial(
        pl.kernel,
        out_type=jax.ShapeDtypeStruct((NC, N, AW), jnp.float32),
        mesh=mesh,
        compiler_params=pltpu.CompilerParams(
            use_tc_tiling_on_sc=False, needs_layout_passes=False),
        scratch_types=[
            pltpu.VMEM_SHARED((N, AW), jnp.float32),
            pltpu.VMEM((B,), jnp.int32),
            pltpu.VMEM((B,), jnp.int32),
            pltpu.VMEM((B, D), jnp.float32),
            pltpu.VMEM((B, 2 * D), jnp.float32),
            pltpu.VMEM((B, D), jnp.float32),
            pltpu.VMEM((B, AW), jnp.float32),
            pltpu.VMEM((D,), jnp.float32),
            pltpu.VMEM((H,), jnp.float32),
            pltpu.SemaphoreType.DMA,
            pltpu.SemaphoreType.DMA,
            pltpu.SemaphoreType.DMA,
        ],
    )
    def sc1(src_hbm, dst_hbm, q_hbm, kv_hbm, ea_hbm, out_hbm,
            acc, sidx, didx, qrows, kvrows, earows, msg, pbuf, axbuf,
            sem_q, sem_kv, sem_ea):
        c = lax.axis_index("c")
        s = lax.axis_index("s")
        wid = s * NC + c

        zero16 = jnp.zeros((16,), jnp.float32)

        def zb(i, carry):
            for j in range(AW // 16):
                msg[i, pl.ds(16 * j, 16)] = zero16
            return carry

        lax.fori_loop(0, B, zb, 0)
        nfull = RPT // B
        for kblk in range(nfull):
            pltpu.sync_copy(msg, acc.at[pl.ds(s * RPT + kblk * B, B)])
        rem = RPT - nfull * B
        if rem:
            pltpu.sync_copy(msg.at[pl.ds(0, rem)],
                            acc.at[pl.ds(s * RPT + nfull * B, rem)])
        plsc.subcore_barrier()

        iota = lax.iota(jnp.int32, 16)
        sumidx = [iota * 8 + cc for cc in range(8)]
        bsel = jnp.where(iota >= 8, 1, 0).astype(jnp.int32)

        def chunk(ch, carry):
            base = wid * EPW + ch * B
            pltpu.sync_copy(src_hbm.at[pl.ds(base, B)], sidx)
            pltpu.sync_copy(dst_hbm.at[pl.ds(base, B)], didx)
            cp_ea = pltpu.async_copy(ea_hbm.at[pl.ds(base, B)], earows, sem_ea)
            cp_q = pltpu.async_copy(q_hbm.at[didx], qrows, sem_q)
            cp_kv = pltpu.async_copy(kv_hbm.at[sidx], kvrows, sem_kv)
            cp_ea.wait()
            cp_q.wait()
            cp_kv.wait()

            def edge(e, ecarry):
                eav = []
                for j in range(8):
                    ea_j = earows[e, pl.ds(16 * j, 16)]
                    eav.append(ea_j)
                    kj = kvrows[e, pl.ds(16 * j, 16)] + ea_j
                    pbuf[pl.ds(16 * j, 16)] = qrows[e, pl.ds(16 * j, 16)] * kj
                al = plsc.load_gather(pbuf, [sumidx[0]])
                for cc in range(1, 8):
                    al = al + plsc.load_gather(pbuf, [sumidx[cc]])
                aexp = jnp.exp(al * _INV_SQRT_C)
                axbuf[...] = aexp
                for j in range(8):
                    bc = plsc.load_gather(axbuf, [bsel + 2 * j])
                    vj = kvrows[e, pl.ds(D + 16 * j, 16)] + eav[j]
                    msg[e, pl.ds(16 * j, 16)] = vj * bc
                msg[e, pl.ds(D, 16)] = aexp
                return ecarry

            lax.fori_loop(0, B, edge, 0)
            pltpu.sync_copy(msg, acc.at[didx], add=True)
            return carry

        lax.fori_loop(0, CHUNKS, chunk, 0)
        plsc.subcore_barrier()
        pltpu.sync_copy(acc.at[pl.ds(s * RPT, RPT)],
                        out_hbm.at[c, pl.ds(s * RPT, RPT)])

    return sc1(src, dst, q, kv, ea)


def _tc_nodes_post(accp, skip, W1a, W1b, K):
    NB = 2000

    def body(acc_ref, s_ref, w1a, w1b, k_ref, xn_ref, a_ref, b_ref):
        accb = acc_ref[...]
        num = accb[0, :, :D] + accb[1, :, :D]
        den = accb[0, :, D:] + accb[1, :, D:]
        inv = 1.0 / (den + 1e-16)
        ratio = num * jnp.dot(inv, k_ref[...], preferred_element_type=jnp.float32)
        out = ratio + s_ref[...]
        xn = _leaky(out)
        xn_ref[...] = xn
        a_ref[...] = jnp.dot(xn, w1a[...], preferred_element_type=jnp.float32)
        b_ref[...] = jnp.dot(xn, w1b[...], preferred_element_type=jnp.float32)

    return pl.pallas_call(
        body,
        grid=(N // NB,),
        in_specs=[
            pl.BlockSpec((NC, NB, AW), lambda i: (0, i, 0)),
            pl.BlockSpec((NB, D), lambda i: (i, 0)),
            pl.BlockSpec((D, DE), lambda i: (0, 0)),
            pl.BlockSpec((D, DE), lambda i: (0, 0)),
            pl.BlockSpec((H, D), lambda i: (0, 0)),
        ],
        out_specs=[
            pl.BlockSpec((NB, D), lambda i: (i, 0)),
            pl.BlockSpec((NB, DE), lambda i: (i, 0)),
            pl.BlockSpec((NB, DE), lambda i: (i, 0)),
        ],
        out_shape=[
            jax.ShapeDtypeStruct((N, D), jnp.float32),
            jax.ShapeDtypeStruct((N, DE), jnp.float32),
            jax.ShapeDtypeStruct((N, DE), jnp.float32),
        ],
    )(accp, skip, W1a, W1b, K)


def _sc_edge_mlp(src, dst, Ap, Bp, ec):
    mesh = plsc.VectorSubcoreMesh(core_axis_name="c", subcore_axis_name="s")

    @functools.partial(
        pl.kernel,
        out_type=jax.ShapeDtypeStruct((E, DE), jnp.float32),
        mesh=mesh,
        compiler_params=pltpu.CompilerParams(
            use_tc_tiling_on_sc=False, needs_layout_passes=False),
        scratch_types=[
            pltpu.VMEM((B,), jnp.int32),
            pltpu.VMEM((B,), jnp.int32),
            pltpu.VMEM((B, DE), jnp.float32),
            pltpu.VMEM((B, DE), jnp.float32),
            pltpu.VMEM((B, DE), jnp.float32),
            pltpu.VMEM((B, DE), jnp.float32),
            pltpu.SemaphoreType.DMA,
            pltpu.SemaphoreType.DMA,
        ],
    )
    def sc2(src_hbm, dst_hbm, a_hbm, b_hbm, ec_hbm, hl_hbm,
            sidx, didx, arows, brows, ecrows, hlrows, sem_a, sem_b):
        c = lax.axis_index("c")
        s = lax.axis_index("s")
        wid = s * NC + c

        def chunk(ch, carry):
            base = wid * EPW + ch * B
            pltpu.sync_copy(src_hbm.at[pl.ds(base, B)], sidx)
            pltpu.sync_copy(dst_hbm.at[pl.ds(base, B)], didx)
            cp_a = pltpu.async_copy(a_hbm.at[sidx], arows, sem_a)
            cp_b = pltpu.async_copy(b_hbm.at[didx], brows, sem_b)
            pltpu.sync_copy(ec_hbm.at[pl.ds(base, B)], ecrows)
            cp_a.wait()
            cp_b.wait()

            def edge(e, ecarry):
                h = (arows[e, pl.ds(0, 16)] + brows[e, pl.ds(0, 16)]
                     + ecrows[e, pl.ds(0, 16)])
                hlrows[e, pl.ds(0, 16)] = jnp.maximum(h, 0.01 * h)
                return ecarry

            lax.fori_loop(0, B, edge, 0)
            pltpu.sync_copy(hlrows, hl_hbm.at[pl.ds(base, B)])
            return carry

        lax.fori_loop(0, CHUNKS, chunk, 0)

    return sc2(src, dst, Ap, Bp, ec)


def _tc_edges_post(hl, edge_attr, W2, b2):
    EB = 4000

    def body(hl_ref, ea_ref, w2, b2_, out_ref):
        h2 = jnp.dot(hl_ref[...], w2[...], preferred_element_type=jnp.float32) + b2_[...]
        out_ref[...] = _leaky(ea_ref[...] + h2)

    return pl.pallas_call(
        body,
        grid=(E // EB,),
        in_specs=[
            pl.BlockSpec((EB, DE), lambda i: (i, 0)),
            pl.BlockSpec((EB, DE), lambda i: (i, 0)),
            pl.BlockSpec((DE, DE), lambda i: (0, 0)),
            pl.BlockSpec((DE,), lambda i: (0,)),
        ],
        out_specs=pl.BlockSpec((EB, DE), lambda i: (i, 0)),
        out_shape=jax.ShapeDtypeStruct((E, DE), jnp.float32),
    )(hl, edge_attr, W2, b2)


def kernel(x, edge_index, edge_attr, batch, Wq, bq, Wk, bk, Wv, bv, We,
           Wskip, bskip, W1, b1, W2, b2):
    src = edge_index[0].astype(jnp.int32)
    dst = edge_index[1].astype(jnp.int32)

    q, kv, skip = _tc_nodes_pre(x, Wq, bq, Wk, bk, Wv, bv, Wskip, bskip)
    ea, ec = _tc_edges_pre(edge_attr, We, W1[2 * D:], b1)
    accp = _sc_attention(src, dst, q, kv, ea)

    K = jnp.asarray(np.kron(np.eye(H), np.ones((1, C))), dtype=jnp.float32)
    x_new, Ap, Bp = _tc_nodes_post(accp, skip, W1[:D], W1[D:2 * D], K)

    hl = _sc_edge_mlp(src, dst, Ap, Bp, ec)
    edge_new = _tc_edges_post(hl, edge_attr, W2, b2)
    return (x_new, edge_new)


# speedup vs baseline: 17.0079x; 1.0730x over previous
import functools

import jax
import jax.numpy as jnp
import numpy as np
from jax import lax
from jax.experimental import pallas as pl
from jax.experimental.pallas import tpu as pltpu
from jax.experimental.pallas import tpu_sc as plsc

N = 10000
E = 320000
D = 128
DE = 16
H = 16
C = 8

NC = 2
NS = 16
NW = NC * NS
EPW = E // NW
B = 32
CHUNKS = EPW // B
TB = EPW - CHUNKS * B
RPT = N // NS
AW = D + H

_INV_SQRT_C = 1.0 / float(np.sqrt(C))


def _leaky(v):
    return jnp.maximum(v, 0.01 * v)


def _tc_nodes_pre(x, Wq, bq, Wk, bk, Wv, bv, Wskip, bskip):
    NB = 2000

    def body(x_ref, wq, bq_, wk, bk_, wv, bv_, ws, bs_, q_ref, k_ref, v_ref, s_ref):
        xb = x_ref[...]
        q_ref[...] = jnp.dot(xb, wq[...], preferred_element_type=jnp.float32) + bq_[...]
        k_ref[...] = jnp.dot(xb, wk[...], preferred_element_type=jnp.float32) + bk_[...]
        v_ref[...] = jnp.dot(xb, wv[...], preferred_element_type=jnp.float32) + bv_[...]
        s_ref[...] = jnp.dot(xb, ws[...], preferred_element_type=jnp.float32) + bs_[...]

    wspec = pl.BlockSpec((D, D), lambda i: (0, 0))
    bspec = pl.BlockSpec((D,), lambda i: (0,))
    nspec = pl.BlockSpec((NB, D), lambda i: (i, 0))
    return pl.pallas_call(
        body,
        grid=(N // NB,),
        in_specs=[
            nspec,
            wspec, bspec, wspec, bspec, wspec, bspec, wspec, bspec,
        ],
        out_specs=[nspec, nspec, nspec, nspec],
        out_shape=[
            jax.ShapeDtypeStruct((N, D), jnp.float32),
            jax.ShapeDtypeStruct((N, D), jnp.float32),
            jax.ShapeDtypeStruct((N, D), jnp.float32),
            jax.ShapeDtypeStruct((N, D), jnp.float32),
        ],
    )(x, Wq, bq, Wk, bk, Wv, bv, Wskip, bskip)


def _tc_edges_pre(edge_attr, We, W1c, b1):
    EB = 4000

    def body(ea_ref, we, w1c, b1_, eaw_ref, ec_ref):
        eb = ea_ref[...]
        eaw_ref[...] = jnp.dot(eb, we[...], preferred_element_type=jnp.float32)
        ec_ref[...] = jnp.dot(eb, w1c[...], preferred_element_type=jnp.float32) + b1_[...]

    return pl.pallas_call(
        body,
        grid=(E // EB,),
        in_specs=[
            pl.BlockSpec((EB, DE), lambda i: (i, 0)),
            pl.BlockSpec((DE, D), lambda i: (0, 0)),
            pl.BlockSpec((DE, DE), lambda i: (0, 0)),
            pl.BlockSpec((DE,), lambda i: (0,)),
        ],
        out_specs=[
            pl.BlockSpec((EB, D), lambda i: (i, 0)),
            pl.BlockSpec((EB, DE), lambda i: (i, 0)),
        ],
        out_shape=[
            jax.ShapeDtypeStruct((E, D), jnp.float32),
            jax.ShapeDtypeStruct((E, DE), jnp.float32),
        ],
    )(edge_attr, We, W1c, b1)


def _sc_attention(ei, q, k, v, ea):
    mesh = plsc.VectorSubcoreMesh(core_axis_name="c", subcore_axis_name="s")

    @functools.partial(
        pl.kernel,
        out_type=[
            jax.ShapeDtypeStruct((NC, N, D), jnp.float32),
            jax.ShapeDtypeStruct((NC, N, H), jnp.float32),
        ],
        mesh=mesh,
        compiler_params=pltpu.CompilerParams(
            use_tc_tiling_on_sc=False, needs_layout_passes=False),
        scratch_types=[
            pltpu.VMEM_SHARED((N, D), jnp.float32),
            pltpu.VMEM_SHARED((N, H), jnp.float32),
            pltpu.VMEM((2, B), jnp.int32),
            pltpu.VMEM((2, B), jnp.int32),
            pltpu.VMEM((B,), jnp.int32),
            pltpu.VMEM((B,), jnp.int32),
            pltpu.VMEM((B, D), jnp.float32),
            pltpu.VMEM((B, D), jnp.float32),
            pltpu.VMEM((B, D), jnp.float32),
            pltpu.VMEM((B, D), jnp.float32),
            pltpu.VMEM((B, D), jnp.float32),
            pltpu.VMEM((B, D), jnp.float32),
            pltpu.VMEM((B, H), jnp.float32),
            pltpu.VMEM((B, H), jnp.float32),
            pltpu.VMEM((B, D), jnp.float32),
            pltpu.SemaphoreType.DMA,
            pltpu.SemaphoreType.DMA,
            pltpu.SemaphoreType.DMA,
            pltpu.SemaphoreType.DMA,
            pltpu.SemaphoreType.DMA,
            pltpu.SemaphoreType.DMA,
        ],
    )
    def sc1(ei_hbm, q_hbm, k_hbm, v_hbm, ea_hbm, outn_hbm, outd_hbm,
            accN, accD, eib0, eib1, didx0, didx1,
            q0, q1, k0, k1, v0, v1, ax0, ax1, earows,
            sem_q, sem_k, sem_v, sem_ea, sem_sc0, sem_sc1):
        c = lax.axis_index("c")
        s = lax.axis_index("s")
        wid = s * NC + c

        zero16 = jnp.zeros((16,), jnp.float32)

        def zb(i, carry):
            for j in range(D // 16):
                k0[i, pl.ds(16 * j, 16)] = zero16
            ax0[i, pl.ds(0, 16)] = zero16
            return carry

        lax.fori_loop(0, B, zb, 0)
        nfull = RPT // B
        for kblk in range(nfull):
            r0 = s * RPT + kblk * B
            pltpu.sync_copy(k0, accN.at[pl.ds(r0, B)])
            pltpu.sync_copy(ax0, accD.at[pl.ds(r0, B)])
        rem = RPT - nfull * B
        if rem:
            r0 = s * RPT + nfull * B
            pltpu.sync_copy(k0.at[pl.ds(0, rem)], accN.at[pl.ds(r0, rem)])
            pltpu.sync_copy(ax0.at[pl.ds(0, rem)], accD.at[pl.ds(r0, rem)])
        plsc.subcore_barrier()

        iota = lax.iota(jnp.int32, 16)
        sumidx = [iota * 8 + cc for cc in range(8)]
        bsel = jnp.where(iota >= 8, 1, 0).astype(jnp.int32)

        slots = [(eib0, didx0, q0, k0, v0, ax0, sem_sc0),
                 (eib1, didx1, q1, k1, v1, ax1, sem_sc1)]

        def base_of(ch):
            return wid * EPW + ch * B

        def issue_g(eib_x, q_slot, k_slot, v_slot):
            pltpu.async_copy(q_hbm.at[eib_x.at[1]], q_slot, sem_q)
            pltpu.async_copy(k_hbm.at[eib_x.at[0]], k_slot, sem_k)
            pltpu.async_copy(v_hbm.at[eib_x.at[0]], v_slot, sem_v)

        def wait_g(eib_x, q_slot, k_slot, v_slot):
            pltpu.make_async_copy(q_hbm.at[eib_x.at[1]], q_slot, sem_q).wait()
            pltpu.make_async_copy(k_hbm.at[eib_x.at[0]], k_slot, sem_k).wait()
            pltpu.make_async_copy(v_hbm.at[eib_x.at[0]], v_slot, sem_v).wait()

        def issue_ea(base):
            pltpu.async_copy(ea_hbm.at[pl.ds(base, B)], earows, sem_ea)

        def wait_ea(base):
            pltpu.make_async_copy(ea_hbm.at[pl.ds(base, B)], earows, sem_ea).wait()

        def issue_sc(k_slot, ax_slot, didx_x, sem):
            pltpu.async_copy(k_slot, accN.at[didx_x], sem, add=True)
            pltpu.async_copy(ax_slot, accD.at[didx_x], sem, add=True)

        def drain_sc(k_slot, ax_slot, didx_x, sem):
            pltpu.make_async_copy(k_slot, accN.at[didx_x], sem).wait()
            pltpu.make_async_copy(ax_slot, accD.at[didx_x], sem).wait()

        def compute_chunk(qr, kr, vr, axr):
            def edge(e, ecarry):
                erow = jnp.full((16,), e, dtype=jnp.int32)
                eav = []
                vv = []
                for j in range(8):
                    ea_j = earows[e, pl.ds(16 * j, 16)]
                    eav.append(ea_j)
                    vv.append(vr[e, pl.ds(16 * j, 16)])
                    kj = kr[e, pl.ds(16 * j, 16)] + ea_j
                    kr[e, pl.ds(16 * j, 16)] = qr[e, pl.ds(16 * j, 16)] * kj
                al = plsc.load_gather(kr, [erow, sumidx[0]])
                for cc in range(1, 8):
                    al = al + plsc.load_gather(kr, [erow, sumidx[cc]])
                aexp = jnp.exp(al * _INV_SQRT_C)
                axr[e, pl.ds(0, 16)] = aexp
                for j in range(8):
                    bc = plsc.load_gather(axr, [erow, bsel + 2 * j])
                    kr[e, pl.ds(16 * j, 16)] = (vv[j] + eav[j]) * bc
                return ecarry

            lax.fori_loop(0, B, edge, 0)

        def body(ch, b, first, last):
            eib_b, didx_b, qb, kb, vb, axb, scb = slots[b]
            bb = base_of(ch)
            pltpu.sync_copy(ei_hbm.at[:, pl.ds(bb, B)], eib_b)
            pltpu.sync_copy(ei_hbm.at[1, pl.ds(bb, B)], didx_b)
            issue_g(eib_b, qb, kb, vb)
            issue_ea(bb)
            wait_ea(bb)
            wait_g(eib_b, qb, kb, vb)
            compute_chunk(qb, kb, vb, axb)
            pltpu.sync_copy(kb, accN.at[didx_b], add=True)
            pltpu.sync_copy(axb, accD.at[didx_b], add=True)

        body(0, 0, first=True, last=False)

        def pair(p, carry):
            ch = 1 + 2 * p
            body(ch, 1, first=False, last=False)
            body(ch + 1, 0, first=False, last=False)
            return carry

        lax.fori_loop(0, (CHUNKS - 2) // 2, pair, 0)
        body(CHUNKS - 1, 1, first=False, last=True)

        base_t = wid * EPW + EPW - B
        DBG_SKIP_TAIL = True
        pltpu.sync_copy(ei_hbm.at[:, pl.ds(base_t, B)], eib0)
        pltpu.sync_copy(ei_hbm.at[1, pl.ds(base_t, B)], didx0)
        issue_g(eib0, q0, k0, v0)
        issue_ea(base_t)
        wait_ea(base_t)
        wait_g(eib0, q0, k0, v0)
        compute_chunk(q0, k0, v0, ax0)
        zero16f = jnp.zeros((16,), jnp.float32)

        def ztail(e, carry):
            for j in range(D // 16):
                k0[e, pl.ds(16 * j, 16)] = zero16f
            ax0[e, pl.ds(0, 16)] = zero16f
            return carry

        lax.fori_loop(0, B - TB, ztail, 0)
        pltpu.sync_copy(k0, accN.at[didx0], add=True)
        pltpu.sync_copy(ax0, accD.at[didx0], add=True)
        plsc.subcore_barrier()
        pltpu.sync_copy(accN.at[pl.ds(s * RPT, RPT)],
                        outn_hbm.at[c, pl.ds(s * RPT, RPT)])
        pltpu.sync_copy(accD.at[pl.ds(s * RPT, RPT)],
                        outd_hbm.at[c, pl.ds(s * RPT, RPT)])

    return sc1(ei, q, k, v, ea)


def _tc_nodes_post(accn, accd, skip, W1a, W1b, K):
    NB = 2000

    def body(an_ref, ad_ref, s_ref, w1a, w1b, k_ref, xn_ref, a_ref, b_ref):
        an = an_ref[...]
        ad = ad_ref[...]
        num = an[0] + an[1]
        den = ad[0] + ad[1]
        inv = 1.0 / (den + 1e-16)
        ratio = num * jnp.dot(inv, k_ref[...], preferred_element_type=jnp.float32)
        out = ratio + s_ref[...]
        xn = _leaky(out)
        xn_ref[...] = xn
        a_ref[...] = jnp.dot(xn, w1a[...], preferred_element_type=jnp.float32)
        b_ref[...] = jnp.dot(xn, w1b[...], preferred_element_type=jnp.float32)

    return pl.pallas_call(
        body,
        grid=(N // NB,),
        in_specs=[
            pl.BlockSpec((NC, NB, D), lambda i: (0, i, 0)),
            pl.BlockSpec((NC, NB, H), lambda i: (0, i, 0)),
            pl.BlockSpec((NB, D), lambda i: (i, 0)),
            pl.BlockSpec((D, DE), lambda i: (0, 0)),
            pl.BlockSpec((D, DE), lambda i: (0, 0)),
            pl.BlockSpec((H, D), lambda i: (0, 0)),
        ],
        out_specs=[
            pl.BlockSpec((NB, D), lambda i: (i, 0)),
            pl.BlockSpec((NB, DE), lambda i: (i, 0)),
            pl.BlockSpec((NB, DE), lambda i: (i, 0)),
        ],
        out_shape=[
            jax.ShapeDtypeStruct((N, D), jnp.float32),
            jax.ShapeDtypeStruct((N, DE), jnp.float32),
            jax.ShapeDtypeStruct((N, DE), jnp.float32),
        ],
    )(accn, accd, skip, W1a, W1b, K)


B2 = 80
CHUNKS2 = EPW // B2


def _sc_edge_mlp(src, dst, Ap, Bp, ec):
    mesh = plsc.VectorSubcoreMesh(core_axis_name="c", subcore_axis_name="s")

    @functools.partial(
        pl.kernel,
        out_type=jax.ShapeDtypeStruct((E, DE), jnp.float32),
        mesh=mesh,
        compiler_params=pltpu.CompilerParams(
            use_tc_tiling_on_sc=False, needs_layout_passes=False),
        scratch_types=[
            pltpu.VMEM((B2,), jnp.int32),
            pltpu.VMEM((B2,), jnp.int32),
            pltpu.VMEM((B2, DE), jnp.float32),
            pltpu.VMEM((B2, DE), jnp.float32),
            pltpu.VMEM((B2, DE), jnp.float32),
            pltpu.VMEM((B2, DE), jnp.float32),
            pltpu.SemaphoreType.DMA,
            pltpu.SemaphoreType.DMA,
        ],
    )
    def sc2(src_hbm, dst_hbm, a_hbm, b_hbm, ec_hbm, hl_hbm,
            sidx, didx, arows, brows, ecrows, hlrows, sem_a, sem_b):
        c = lax.axis_index("c")
        s = lax.axis_index("s")
        wid = s * NC + c

        def chunk(ch, carry):
            base = wid * EPW + ch * B2
            pltpu.sync_copy(src_hbm.at[pl.ds(base, B2)], sidx)
            pltpu.sync_copy(dst_hbm.at[pl.ds(base, B2)], didx)
            cp_a = pltpu.async_copy(a_hbm.at[sidx], arows, sem_a)
            cp_b = pltpu.async_copy(b_hbm.at[didx], brows, sem_b)
            pltpu.sync_copy(ec_hbm.at[pl.ds(base, B2)], ecrows)
            cp_a.wait()
            cp_b.wait()

            def edge(e, ecarry):
                h = (arows[e, pl.ds(0, 16)] + brows[e, pl.ds(0, 16)]
                     + ecrows[e, pl.ds(0, 16)])
                hlrows[e, pl.ds(0, 16)] = jnp.maximum(h, 0.01 * h)
                return ecarry

            lax.fori_loop(0, B2, edge, 0)
            pltpu.sync_copy(hlrows, hl_hbm.at[pl.ds(base, B2)])
            return carry

        lax.fori_loop(0, CHUNKS2, chunk, 0)

    return sc2(src, dst, Ap, Bp, ec)


def _tc_edges_post(hl, edge_attr, W2, b2):
    EB = 4000

    def body(hl_ref, ea_ref, w2, b2_, out_ref):
        h2 = jnp.dot(hl_ref[...], w2[...], preferred_element_type=jnp.float32) + b2_[...]
        out_ref[...] = _leaky(ea_ref[...] + h2)

    return pl.pallas_call(
        body,
        grid=(E // EB,),
        in_specs=[
            pl.BlockSpec((EB, DE), lambda i: (i, 0)),
            pl.BlockSpec((EB, DE), lambda i: (i, 0)),
            pl.BlockSpec((DE, DE), lambda i: (0, 0)),
            pl.BlockSpec((DE,), lambda i: (0,)),
        ],
        out_specs=pl.BlockSpec((EB, DE), lambda i: (i, 0)),
        out_shape=jax.ShapeDtypeStruct((E, DE), jnp.float32),
    )(hl, edge_attr, W2, b2)


def kernel(x, edge_index, edge_attr, batch, Wq, bq, Wk, bk, Wv, bv, We,
           Wskip, bskip, W1, b1, W2, b2):
    ei = edge_index.astype(jnp.int32)
    src = ei[0]
    dst = ei[1]

    q, k, v, skip = _tc_nodes_pre(x, Wq, bq, Wk, bk, Wv, bv, Wskip, bskip)
    ea, ec = _tc_edges_pre(edge_attr, We, W1[2 * D:], b1)
    accn, accd = _sc_attention(ei, q, k, v, ea)

    K = jnp.asarray(np.kron(np.eye(H), np.ones((1, C))), dtype=jnp.float32)
    x_new, Ap, Bp = _tc_nodes_post(accn, accd, skip, W1[:D], W1[D:2 * D], K)

    hl = _sc_edge_mlp(src, dst, Ap, Bp, ec)
    edge_new = _tc_edges_post(hl, edge_attr, W2, b2)
    return (x_new, edge_new)


# speedup vs baseline: 17.5712x; 1.0331x over previous
import functools

import jax
import jax.numpy as jnp
import numpy as np
from jax import lax
from jax.experimental import pallas as pl
from jax.experimental.pallas import tpu as pltpu
from jax.experimental.pallas import tpu_sc as plsc

N = 10000
E = 320000
D = 128
DE = 16
H = 16
C = 8

NC = 2
NS = 16
NW = NC * NS
EPW = E // NW
B = 32
CHUNKS = EPW // B
TB = EPW - CHUNKS * B
RPT = N // NS
AW = D + H

_INV_SQRT_C = 1.0 / float(np.sqrt(C))


def _leaky(v):
    return jnp.maximum(v, 0.01 * v)


def _tc_nodes_pre(x, Wq, bq, Wk, bk, Wv, bv, Wskip, bskip):
    NB = 2000

    def body(x_ref, wq, bq_, wk, bk_, wv, bv_, ws, bs_, q_ref, k_ref, v_ref, s_ref):
        xb = x_ref[...]
        q_ref[...] = jnp.dot(xb, wq[...], preferred_element_type=jnp.float32) + bq_[...]
        k_ref[...] = jnp.dot(xb, wk[...], preferred_element_type=jnp.float32) + bk_[...]
        v_ref[...] = jnp.dot(xb, wv[...], preferred_element_type=jnp.float32) + bv_[...]
        s_ref[...] = jnp.dot(xb, ws[...], preferred_element_type=jnp.float32) + bs_[...]

    wspec = pl.BlockSpec((D, D), lambda i: (0, 0))
    bspec = pl.BlockSpec((D,), lambda i: (0,))
    nspec = pl.BlockSpec((NB, D), lambda i: (i, 0))
    return pl.pallas_call(
        body,
        grid=(N // NB,),
        in_specs=[
            nspec,
            wspec, bspec, wspec, bspec, wspec, bspec, wspec, bspec,
        ],
        out_specs=[nspec, nspec, nspec, nspec],
        out_shape=[
            jax.ShapeDtypeStruct((N, D), jnp.float32),
            jax.ShapeDtypeStruct((N, D), jnp.float32),
            jax.ShapeDtypeStruct((N, D), jnp.float32),
            jax.ShapeDtypeStruct((N, D), jnp.float32),
        ],
    )(x, Wq, bq, Wk, bk, Wv, bv, Wskip, bskip)


def _tc_edges_pre(edge_attr, We, W1c, b1):
    EB = 4000

    def body(ea_ref, we, w1c, b1_, eaw_ref, ec_ref):
        eb = ea_ref[...]
        eaw_ref[...] = jnp.dot(eb, we[...], preferred_element_type=jnp.float32)
        ec_ref[...] = jnp.dot(eb, w1c[...], preferred_element_type=jnp.float32) + b1_[...]

    return pl.pallas_call(
        body,
        grid=(E // EB,),
        in_specs=[
            pl.BlockSpec((EB, DE), lambda i: (i, 0)),
            pl.BlockSpec((DE, D), lambda i: (0, 0)),
            pl.BlockSpec((DE, DE), lambda i: (0, 0)),
            pl.BlockSpec((DE,), lambda i: (0,)),
        ],
        out_specs=[
            pl.BlockSpec((EB, D), lambda i: (i, 0)),
            pl.BlockSpec((EB, DE), lambda i: (i, 0)),
        ],
        out_shape=[
            jax.ShapeDtypeStruct((E, D), jnp.float32),
            jax.ShapeDtypeStruct((E, DE), jnp.float32),
        ],
    )(edge_attr, We, W1c, b1)


def _sc_attention(ei, q, k, v, ea):
    mesh = plsc.VectorSubcoreMesh(core_axis_name="c", subcore_axis_name="s")

    @functools.partial(
        pl.kernel,
        out_type=[
            jax.ShapeDtypeStruct((NC, N, D), jnp.float32),
            jax.ShapeDtypeStruct((NC, N, H), jnp.float32),
        ],
        mesh=mesh,
        compiler_params=pltpu.CompilerParams(
            use_tc_tiling_on_sc=False, needs_layout_passes=False),
        scratch_types=[
            pltpu.VMEM_SHARED((N, D), jnp.float32),
            pltpu.VMEM_SHARED((N, H), jnp.float32),
            pltpu.VMEM((2, B), jnp.int32),
            pltpu.VMEM((2, B), jnp.int32),
            pltpu.VMEM((B,), jnp.int32),
            pltpu.VMEM((B,), jnp.int32),
            pltpu.VMEM((B, D), jnp.float32),
            pltpu.VMEM((B, D), jnp.float32),
            pltpu.VMEM((B, D), jnp.float32),
            pltpu.VMEM((B, D), jnp.float32),
            pltpu.VMEM((B, D), jnp.float32),
            pltpu.VMEM((B, D), jnp.float32),
            pltpu.VMEM((B, H), jnp.float32),
            pltpu.VMEM((B, H), jnp.float32),
            pltpu.VMEM((B, D), jnp.float32),
            pltpu.SemaphoreType.DMA,
            pltpu.SemaphoreType.DMA,
            pltpu.SemaphoreType.DMA,
            pltpu.SemaphoreType.DMA,
            pltpu.SemaphoreType.DMA,
            pltpu.SemaphoreType.DMA,
        ],
    )
    def sc1(ei_hbm, q_hbm, k_hbm, v_hbm, ea_hbm, outn_hbm, outd_hbm,
            accN, accD, eib0, eib1, didx0, didx1,
            q0, q1, k0, k1, v0, v1, ax0, ax1, earows,
            sem_q, sem_k, sem_v, sem_ea, sem_sc0, sem_sc1):
        c = lax.axis_index("c")
        s = lax.axis_index("s")
        wid = s * NC + c

        zero16 = jnp.zeros((16,), jnp.float32)

        def zb(i, carry):
            for j in range(D // 16):
                k0[i, pl.ds(16 * j, 16)] = zero16
            ax0[i, pl.ds(0, 16)] = zero16
            return carry

        lax.fori_loop(0, B, zb, 0)
        nfull = RPT // B
        for kblk in range(nfull):
            r0 = s * RPT + kblk * B
            pltpu.sync_copy(k0, accN.at[pl.ds(r0, B)])
            pltpu.sync_copy(ax0, accD.at[pl.ds(r0, B)])
        rem = RPT - nfull * B
        if rem:
            r0 = s * RPT + nfull * B
            pltpu.sync_copy(k0.at[pl.ds(0, rem)], accN.at[pl.ds(r0, rem)])
            pltpu.sync_copy(ax0.at[pl.ds(0, rem)], accD.at[pl.ds(r0, rem)])
        plsc.subcore_barrier()

        iota = lax.iota(jnp.int32, 16)
        sumidx = [iota * 8 + cc for cc in range(8)]
        bsel = jnp.where(iota >= 8, 1, 0).astype(jnp.int32)

        slots = [(eib0, didx0, q0, k0, v0, ax0, sem_sc0),
                 (eib1, didx1, q1, k1, v1, ax1, sem_sc1)]

        def base_of(ch):
            return wid * EPW + ch * B

        def issue_g(eib_x, q_slot, k_slot, v_slot):
            pltpu.async_copy(q_hbm.at[eib_x.at[1]], q_slot, sem_q)
            pltpu.async_copy(k_hbm.at[eib_x.at[0]], k_slot, sem_k)
            pltpu.async_copy(v_hbm.at[eib_x.at[0]], v_slot, sem_v)

        def wait_g(eib_x, q_slot, k_slot, v_slot):
            pltpu.make_async_copy(q_hbm.at[eib_x.at[1]], q_slot, sem_q).wait()
            pltpu.make_async_copy(k_hbm.at[eib_x.at[0]], k_slot, sem_k).wait()
            pltpu.make_async_copy(v_hbm.at[eib_x.at[0]], v_slot, sem_v).wait()

        def issue_ea(base):
            pltpu.async_copy(ea_hbm.at[pl.ds(base, B)], earows, sem_ea)

        def wait_ea(base):
            pltpu.make_async_copy(ea_hbm.at[pl.ds(base, B)], earows, sem_ea).wait()

        def issue_sc(k_slot, ax_slot, didx_x, sem):
            pltpu.async_copy(k_slot, accN.at[didx_x], sem, add=True)
            pltpu.async_copy(ax_slot, accD.at[didx_x], sem, add=True)

        def drain_sc(k_slot, ax_slot, didx_x, sem):
            pltpu.make_async_copy(k_slot, accN.at[didx_x], sem).wait()
            pltpu.make_async_copy(ax_slot, accD.at[didx_x], sem).wait()

        def compute_chunk(qr, kr, vr, axr):
            def edge(e, ecarry):
                erow = jnp.full((16,), e, dtype=jnp.int32)
                eav = []
                vv = []
                for j in range(8):
                    ea_j = earows[e, pl.ds(16 * j, 16)]
                    eav.append(ea_j)
                    vv.append(vr[e, pl.ds(16 * j, 16)])
                    kj = kr[e, pl.ds(16 * j, 16)] + ea_j
                    kr[e, pl.ds(16 * j, 16)] = qr[e, pl.ds(16 * j, 16)] * kj
                al = plsc.load_gather(kr, [erow, sumidx[0]])
                for cc in range(1, 8):
                    al = al + plsc.load_gather(kr, [erow, sumidx[cc]])
                aexp = jnp.exp(al * _INV_SQRT_C)
                axr[e, pl.ds(0, 16)] = aexp
                for j in range(8):
                    bc = plsc.load_gather(axr, [erow, bsel + 2 * j])
                    kr[e, pl.ds(16 * j, 16)] = (vv[j] + eav[j]) * bc
                return ecarry

            lax.fori_loop(0, B, edge, 0)

        def body(ch, b, first, last):
            eib_b, didx_b, qb, kb, vb, axb, scb = slots[b]
            eib_n, didx_n, qn, kn, vn, axn, scn = slots[1 - b]
            wait_ea(base_of(ch))
            wait_g(eib_b, qb, kb, vb)
            compute_chunk(qb, kb, vb, axb)
            issue_sc(kb, axb, didx_b, scb)
            if not first:
                drain_sc(kn, axn, didx_n, scn)
            if not last:
                nb = base_of(ch + 1)
                pltpu.sync_copy(ei_hbm.at[:, pl.ds(nb, B)], eib_n)
                pltpu.sync_copy(ei_hbm.at[1, pl.ds(nb, B)], didx_n)
                issue_g(eib_n, qn, kn, vn)
                issue_ea(nb)

        base0 = base_of(0)
        pltpu.sync_copy(ei_hbm.at[:, pl.ds(base0, B)], eib0)
        pltpu.sync_copy(ei_hbm.at[1, pl.ds(base0, B)], didx0)
        issue_g(eib0, q0, k0, v0)
        issue_ea(base0)
        body(0, 0, first=True, last=False)

        def pair(p, carry):
            ch = 1 + 2 * p
            body(ch, 1, first=False, last=False)
            body(ch + 1, 0, first=False, last=False)
            return carry

        lax.fori_loop(0, (CHUNKS - 2) // 2, pair, 0)
        body(CHUNKS - 1, 1, first=False, last=True)
        drain_sc(k1, ax1, didx1, sem_sc1)

        base_t = wid * EPW + EPW - B
        pltpu.sync_copy(ei_hbm.at[:, pl.ds(base_t, B)], eib0)
        pltpu.sync_copy(ei_hbm.at[1, pl.ds(base_t, B)], didx0)
        issue_g(eib0, q0, k0, v0)
        issue_ea(base_t)
        wait_ea(base_t)
        wait_g(eib0, q0, k0, v0)
        compute_chunk(q0, k0, v0, ax0)
        zero16f = jnp.zeros((16,), jnp.float32)

        def ztail(e, carry):
            for j in range(D // 16):
                k0[e, pl.ds(16 * j, 16)] = zero16f
            ax0[e, pl.ds(0, 16)] = zero16f
            return carry

        lax.fori_loop(0, B - TB, ztail, 0)
        pltpu.sync_copy(k0, accN.at[didx0], add=True)
        pltpu.sync_copy(ax0, accD.at[didx0], add=True)
        plsc.subcore_barrier()
        pltpu.sync_copy(accN.at[pl.ds(s * RPT, RPT)],
                        outn_hbm.at[c, pl.ds(s * RPT, RPT)])
        pltpu.sync_copy(accD.at[pl.ds(s * RPT, RPT)],
                        outd_hbm.at[c, pl.ds(s * RPT, RPT)])

    return sc1(ei, q, k, v, ea)


def _tc_nodes_post(accn, accd, skip, W1a, W1b, K):
    NB = 2000

    def body(an_ref, ad_ref, s_ref, w1a, w1b, k_ref, xn_ref, a_ref, b_ref):
        an = an_ref[...]
        ad = ad_ref[...]
        num = an[0] + an[1]
        den = ad[0] + ad[1]
        inv = 1.0 / (den + 1e-16)
        ratio = num * jnp.dot(inv, k_ref[...], preferred_element_type=jnp.float32)
        out = ratio + s_ref[...]
        xn = _leaky(out)
        xn_ref[...] = xn
        a_ref[...] = jnp.dot(xn, w1a[...], preferred_element_type=jnp.float32)
        b_ref[...] = jnp.dot(xn, w1b[...], preferred_element_type=jnp.float32)

    return pl.pallas_call(
        body,
        grid=(N // NB,),
        in_specs=[
            pl.BlockSpec((NC, NB, D), lambda i: (0, i, 0)),
            pl.BlockSpec((NC, NB, H), lambda i: (0, i, 0)),
            pl.BlockSpec((NB, D), lambda i: (i, 0)),
            pl.BlockSpec((D, DE), lambda i: (0, 0)),
            pl.BlockSpec((D, DE), lambda i: (0, 0)),
            pl.BlockSpec((H, D), lambda i: (0, 0)),
        ],
        out_specs=[
            pl.BlockSpec((NB, D), lambda i: (i, 0)),
            pl.BlockSpec((NB, DE), lambda i: (i, 0)),
            pl.BlockSpec((NB, DE), lambda i: (i, 0)),
        ],
        out_shape=[
            jax.ShapeDtypeStruct((N, D), jnp.float32),
            jax.ShapeDtypeStruct((N, DE), jnp.float32),
            jax.ShapeDtypeStruct((N, DE), jnp.float32),
        ],
    )(accn, accd, skip, W1a, W1b, K)


B2 = 80
CHUNKS2 = EPW // B2


def _sc_edge_mlp(src, dst, Ap, Bp, ec):
    mesh = plsc.VectorSubcoreMesh(core_axis_name="c", subcore_axis_name="s")

    @functools.partial(
        pl.kernel,
        out_type=jax.ShapeDtypeStruct((E, DE), jnp.float32),
        mesh=mesh,
        compiler_params=pltpu.CompilerParams(
            use_tc_tiling_on_sc=False, needs_layout_passes=False),
        scratch_types=[
            pltpu.VMEM((B2,), jnp.int32),
            pltpu.VMEM((B2,), jnp.int32),
            pltpu.VMEM((B2, DE), jnp.float32),
            pltpu.VMEM((B2, DE), jnp.float32),
            pltpu.VMEM((B2, DE), jnp.float32),
            pltpu.VMEM((B2, DE), jnp.float32),
            pltpu.SemaphoreType.DMA,
            pltpu.SemaphoreType.DMA,
        ],
    )
    def sc2(src_hbm, dst_hbm, a_hbm, b_hbm, ec_hbm, hl_hbm,
            sidx, didx, arows, brows, ecrows, hlrows, sem_a, sem_b):
        c = lax.axis_index("c")
        s = lax.axis_index("s")
        wid = s * NC + c

        def chunk(ch, carry):
            base = wid * EPW + ch * B2
            pltpu.sync_copy(src_hbm.at[pl.ds(base, B2)], sidx)
            pltpu.sync_copy(dst_hbm.at[pl.ds(base, B2)], didx)
            cp_a = pltpu.async_copy(a_hbm.at[sidx], arows, sem_a)
            cp_b = pltpu.async_copy(b_hbm.at[didx], brows, sem_b)
            pltpu.sync_copy(ec_hbm.at[pl.ds(base, B2)], ecrows)
            cp_a.wait()
            cp_b.wait()

            def edge(e, ecarry):
                h = (arows[e, pl.ds(0, 16)] + brows[e, pl.ds(0, 16)]
                     + ecrows[e, pl.ds(0, 16)])
                hlrows[e, pl.ds(0, 16)] = jnp.maximum(h, 0.01 * h)
                return ecarry

            lax.fori_loop(0, B2, edge, 0)
            pltpu.sync_copy(hlrows, hl_hbm.at[pl.ds(base, B2)])
            return carry

        lax.fori_loop(0, CHUNKS2, chunk, 0)

    return sc2(src, dst, Ap, Bp, ec)


def _tc_edges_post(hl, edge_attr, W2, b2):
    EB = 4000

    def body(hl_ref, ea_ref, w2, b2_, out_ref):
        h2 = jnp.dot(hl_ref[...], w2[...], preferred_element_type=jnp.float32) + b2_[...]
        out_ref[...] = _leaky(ea_ref[...] + h2)

    return pl.pallas_call(
        body,
        grid=(E // EB,),
        in_specs=[
            pl.BlockSpec((EB, DE), lambda i: (i, 0)),
            pl.BlockSpec((EB, DE), lambda i: (i, 0)),
            pl.BlockSpec((DE, DE), lambda i: (0, 0)),
            pl.BlockSpec((DE,), lambda i: (0,)),
        ],
        out_specs=pl.BlockSpec((EB, DE), lambda i: (i, 0)),
        out_shape=jax.ShapeDtypeStruct((E, DE), jnp.float32),
    )(hl, edge_attr, W2, b2)


def kernel(x, edge_index, edge_attr, batch, Wq, bq, Wk, bk, Wv, bv, We,
           Wskip, bskip, W1, b1, W2, b2):
    ei = edge_index.astype(jnp.int32)
    src = ei[0]
    dst = ei[1]

    q, k, v, skip = _tc_nodes_pre(x, Wq, bq, Wk, bk, Wv, bv, Wskip, bskip)
    ea, ec = _tc_edges_pre(edge_attr, We, W1[2 * D:], b1)
    accn, accd = _sc_attention(ei, q, k, v, ea)

    K = jnp.asarray(np.kron(np.eye(H), np.ones((1, C))), dtype=jnp.float32)
    x_new, Ap, Bp = _tc_nodes_post(accn, accd, skip, W1[:D], W1[D:2 * D], K)

    hl = _sc_edge_mlp(src, dst, Ap, Bp, ec)
    edge_new = _tc_edges_post(hl, edge_attr, W2, b2)
    return (x_new, edge_new)


# speedup vs baseline: 22.7025x; 1.2920x over previous
import functools

import jax
import jax.numpy as jnp
import numpy as np
from jax import lax
from jax.experimental import pallas as pl
from jax.experimental.pallas import tpu as pltpu
from jax.experimental.pallas import tpu_sc as plsc

N = 10000
E = 320000
D = 128
DE = 16
H = 16
C = 8

NC = 2
NS = 16
NW = NC * NS
EPW = E // NW
B = 32
CHUNKS = EPW // B
TB = EPW - CHUNKS * B
RPT = N // NS
AW = D + H

_INV_SQRT_C = 1.0 / float(np.sqrt(C))


def _leaky(v):
    return jnp.maximum(v, 0.01 * v)


_GD = lax.GatherDimensionNumbers(offset_dims=(), collapsed_slice_dims=(0,),
                                 start_index_map=(0,))


def _vtake(x, idx):
    return lax.gather(x, idx[:, None], _GD, (1,),
                      mode=lax.GatherScatterMode.PROMISE_IN_BOUNDS)


def _tc_nodes_pre(x, Wq, bq, Wk, bk, Wv, bv, Wskip, bskip):
    NB = 2000

    def body(x_ref, wq, bq_, wk, bk_, wv, bv_, ws, bs_, q_ref, k_ref, v_ref, s_ref):
        xb = x_ref[...]
        q_ref[...] = jnp.dot(xb, wq[...], preferred_element_type=jnp.float32) + bq_[...]
        k_ref[...] = jnp.dot(xb, wk[...], preferred_element_type=jnp.float32) + bk_[...]
        v_ref[...] = jnp.dot(xb, wv[...], preferred_element_type=jnp.float32) + bv_[...]
        s_ref[...] = jnp.dot(xb, ws[...], preferred_element_type=jnp.float32) + bs_[...]

    wspec = pl.BlockSpec((D, D), lambda i: (0, 0))
    bspec = pl.BlockSpec((D,), lambda i: (0,))
    nspec = pl.BlockSpec((NB, D), lambda i: (i, 0))
    return pl.pallas_call(
        body,
        grid=(N // NB,),
        in_specs=[
            nspec,
            wspec, bspec, wspec, bspec, wspec, bspec, wspec, bspec,
        ],
        out_specs=[nspec, nspec, nspec, nspec],
        out_shape=[
            jax.ShapeDtypeStruct((N, D), jnp.float32),
            jax.ShapeDtypeStruct((N, D), jnp.float32),
            jax.ShapeDtypeStruct((N, D), jnp.float32),
            jax.ShapeDtypeStruct((N, D), jnp.float32),
        ],
    )(x, Wq, bq, Wk, bk, Wv, bv, Wskip, bskip)


def _tc_edges_pre(edge_attr, We, W1c, b1):
    EB = 4000

    def body(ea_ref, we, w1c, b1_, eaw_ref, ec_ref):
        eb = ea_ref[...]
        eaw_ref[...] = jnp.dot(eb, we[...], preferred_element_type=jnp.float32)
        ec_ref[...] = jnp.dot(eb, w1c[...], preferred_element_type=jnp.float32) + b1_[...]

    return pl.pallas_call(
        body,
        grid=(E // EB,),
        in_specs=[
            pl.BlockSpec((EB, DE), lambda i: (i, 0)),
            pl.BlockSpec((DE, D), lambda i: (0, 0)),
            pl.BlockSpec((DE, DE), lambda i: (0, 0)),
            pl.BlockSpec((DE,), lambda i: (0,)),
        ],
        out_specs=[
            pl.BlockSpec((EB, D), lambda i: (i, 0)),
            pl.BlockSpec((EB, DE), lambda i: (i, 0)),
        ],
        out_shape=[
            jax.ShapeDtypeStruct((E, D), jnp.float32),
            jax.ShapeDtypeStruct((E, DE), jnp.float32),
        ],
    )(edge_attr, We, W1c, b1)


def _sc_attention(ei, q, k, v, ea):
    mesh = plsc.VectorSubcoreMesh(core_axis_name="c", subcore_axis_name="s")

    @functools.partial(
        pl.kernel,
        out_type=[
            jax.ShapeDtypeStruct((NC, N, D), jnp.float32),
            jax.ShapeDtypeStruct((NC, N, H), jnp.float32),
        ],
        mesh=mesh,
        compiler_params=pltpu.CompilerParams(
            use_tc_tiling_on_sc=False, needs_layout_passes=False),
        scratch_types=[
            pltpu.VMEM_SHARED((N, D), jnp.float32),
            pltpu.VMEM_SHARED((N, H), jnp.float32),
            pltpu.VMEM((2, B), jnp.int32),
            pltpu.VMEM((2, B), jnp.int32),
            pltpu.VMEM((B,), jnp.int32),
            pltpu.VMEM((B,), jnp.int32),
            pltpu.VMEM((B, D), jnp.float32),
            pltpu.VMEM((B, D), jnp.float32),
            pltpu.VMEM((B, D), jnp.float32),
            pltpu.VMEM((B, D), jnp.float32),
            pltpu.VMEM((B, D), jnp.float32),
            pltpu.VMEM((B, D), jnp.float32),
            pltpu.VMEM((B, H), jnp.float32),
            pltpu.VMEM((B, H), jnp.float32),
            pltpu.VMEM((B, D), jnp.float32),
            pltpu.SemaphoreType.DMA,
            pltpu.SemaphoreType.DMA,
            pltpu.SemaphoreType.DMA,
            pltpu.SemaphoreType.DMA,
            pltpu.SemaphoreType.DMA,
            pltpu.SemaphoreType.DMA,
        ],
    )
    def sc1(ei_hbm, q_hbm, k_hbm, v_hbm, ea_hbm, outn_hbm, outd_hbm,
            accN, accD, eib0, eib1, didx0, didx1,
            q0, q1, k0, k1, v0, v1, ax0, ax1, earows,
            sem_q, sem_k, sem_v, sem_ea, sem_sc0, sem_sc1):
        c = lax.axis_index("c")
        s = lax.axis_index("s")
        wid = s * NC + c

        zero16 = jnp.zeros((16,), jnp.float32)

        def zb(i, carry):
            for j in range(D // 16):
                k0[i, pl.ds(16 * j, 16)] = zero16
            ax0[i, pl.ds(0, 16)] = zero16
            return carry

        lax.fori_loop(0, B, zb, 0)
        nfull = RPT // B
        for kblk in range(nfull):
            r0 = s * RPT + kblk * B
            pltpu.sync_copy(k0, accN.at[pl.ds(r0, B)])
            pltpu.sync_copy(ax0, accD.at[pl.ds(r0, B)])
        rem = RPT - nfull * B
        if rem:
            r0 = s * RPT + nfull * B
            pltpu.sync_copy(k0.at[pl.ds(0, rem)], accN.at[pl.ds(r0, rem)])
            pltpu.sync_copy(ax0.at[pl.ds(0, rem)], accD.at[pl.ds(r0, rem)])
        plsc.subcore_barrier()

        iota = lax.iota(jnp.int32, 16)
        sumidx = [iota * 8 + cc for cc in range(8)]
        bsel = jnp.where(iota >= 8, 1, 0).astype(jnp.int32)

        slots = [(eib0, didx0, q0, k0, v0, ax0, sem_sc0),
                 (eib1, didx1, q1, k1, v1, ax1, sem_sc1)]

        def base_of(ch):
            return wid * EPW + ch * B

        def issue_g(eib_x, q_slot, k_slot, v_slot):
            pltpu.async_copy(q_hbm.at[eib_x.at[1]], q_slot, sem_q)
            pltpu.async_copy(k_hbm.at[eib_x.at[0]], k_slot, sem_k)
            pltpu.async_copy(v_hbm.at[eib_x.at[0]], v_slot, sem_v)

        def wait_g(eib_x, q_slot, k_slot, v_slot):
            pltpu.make_async_copy(q_hbm.at[eib_x.at[1]], q_slot, sem_q).wait()
            pltpu.make_async_copy(k_hbm.at[eib_x.at[0]], k_slot, sem_k).wait()
            pltpu.make_async_copy(v_hbm.at[eib_x.at[0]], v_slot, sem_v).wait()

        def issue_ea(base):
            pltpu.async_copy(ea_hbm.at[pl.ds(base, B)], earows, sem_ea)

        def wait_ea(base):
            pltpu.make_async_copy(ea_hbm.at[pl.ds(base, B)], earows, sem_ea).wait()

        def issue_sc(k_slot, ax_slot, didx_x, sem):
            pltpu.async_copy(k_slot, accN.at[didx_x], sem, add=True)
            pltpu.async_copy(ax_slot, accD.at[didx_x], sem, add=True)

        def drain_sc(k_slot, ax_slot, didx_x, sem):
            pltpu.make_async_copy(k_slot, accN.at[didx_x], sem).wait()
            pltpu.make_async_copy(ax_slot, accD.at[didx_x], sem).wait()

        def compute_chunk(qr, kr, vr, axr):
            @plsc.parallel_loop(0, B, unroll=2)
            def edge(e):
                erow = jnp.full((16,), e, dtype=jnp.int32)
                eav = []
                vv = []
                for j in range(8):
                    ea_j = earows[e, pl.ds(16 * j, 16)]
                    eav.append(ea_j)
                    vv.append(vr[e, pl.ds(16 * j, 16)])
                    kj = kr[e, pl.ds(16 * j, 16)] + ea_j
                    kr[e, pl.ds(16 * j, 16)] = qr[e, pl.ds(16 * j, 16)] * kj
                al = plsc.load_gather(kr, [erow, sumidx[0]])
                for cc in range(1, 8):
                    al = al + plsc.load_gather(kr, [erow, sumidx[cc]])
                aexp = jnp.exp(al * _INV_SQRT_C)
                axr[e, pl.ds(0, 16)] = aexp
                for j in range(8):
                    bc = _vtake(aexp, bsel + 2 * j)
                    kr[e, pl.ds(16 * j, 16)] = (vv[j] + eav[j]) * bc

        def body(ch, b, first, last):
            eib_b, didx_b, qb, kb, vb, axb, scb = slots[b]
            eib_n, didx_n, qn, kn, vn, axn, scn = slots[1 - b]
            wait_ea(base_of(ch))
            wait_g(eib_b, qb, kb, vb)
            compute_chunk(qb, kb, vb, axb)
            issue_sc(kb, axb, didx_b, scb)
            if not first:
                drain_sc(kn, axn, didx_n, scn)
            if not last:
                nb = base_of(ch + 1)
                pltpu.sync_copy(ei_hbm.at[:, pl.ds(nb, B)], eib_n)
                pltpu.sync_copy(ei_hbm.at[1, pl.ds(nb, B)], didx_n)
                issue_g(eib_n, qn, kn, vn)
                issue_ea(nb)

        base0 = base_of(0)
        pltpu.sync_copy(ei_hbm.at[:, pl.ds(base0, B)], eib0)
        pltpu.sync_copy(ei_hbm.at[1, pl.ds(base0, B)], didx0)
        issue_g(eib0, q0, k0, v0)
        issue_ea(base0)
        body(0, 0, first=True, last=False)

        def pair(p, carry):
            ch = 1 + 2 * p
            body(ch, 1, first=False, last=False)
            body(ch + 1, 0, first=False, last=False)
            return carry

        lax.fori_loop(0, (CHUNKS - 2) // 2, pair, 0)
        body(CHUNKS - 1, 1, first=False, last=True)
        drain_sc(k1, ax1, didx1, sem_sc1)

        base_t = wid * EPW + EPW - B
        pltpu.sync_copy(ei_hbm.at[:, pl.ds(base_t, B)], eib0)
        pltpu.sync_copy(ei_hbm.at[1, pl.ds(base_t, B)], didx0)
        issue_g(eib0, q0, k0, v0)
        issue_ea(base_t)
        wait_ea(base_t)
        wait_g(eib0, q0, k0, v0)
        compute_chunk(q0, k0, v0, ax0)
        zero16f = jnp.zeros((16,), jnp.float32)

        def ztail(e, carry):
            for j in range(D // 16):
                k0[e, pl.ds(16 * j, 16)] = zero16f
            ax0[e, pl.ds(0, 16)] = zero16f
            return carry

        lax.fori_loop(0, B - TB, ztail, 0)
        pltpu.sync_copy(k0, accN.at[didx0], add=True)
        pltpu.sync_copy(ax0, accD.at[didx0], add=True)
        plsc.subcore_barrier()
        pltpu.sync_copy(accN.at[pl.ds(s * RPT, RPT)],
                        outn_hbm.at[c, pl.ds(s * RPT, RPT)])
        pltpu.sync_copy(accD.at[pl.ds(s * RPT, RPT)],
                        outd_hbm.at[c, pl.ds(s * RPT, RPT)])

    return sc1(ei, q, k, v, ea)


def _tc_nodes_post(accn, accd, skip, W1a, W1b, K):
    NB = 2000

    def body(an_ref, ad_ref, s_ref, w1a, w1b, k_ref, xn_ref, a_ref, b_ref):
        an = an_ref[...]
        ad = ad_ref[...]
        num = an[0] + an[1]
        den = ad[0] + ad[1]
        inv = 1.0 / (den + 1e-16)
        ratio = num * jnp.dot(inv, k_ref[...], preferred_element_type=jnp.float32)
        out = ratio + s_ref[...]
        xn = _leaky(out)
        xn_ref[...] = xn
        a_ref[...] = jnp.dot(xn, w1a[...], preferred_element_type=jnp.float32)
        b_ref[...] = jnp.dot(xn, w1b[...], preferred_element_type=jnp.float32)

    return pl.pallas_call(
        body,
        grid=(N // NB,),
        in_specs=[
            pl.BlockSpec((NC, NB, D), lambda i: (0, i, 0)),
            pl.BlockSpec((NC, NB, H), lambda i: (0, i, 0)),
            pl.BlockSpec((NB, D), lambda i: (i, 0)),
            pl.BlockSpec((D, DE), lambda i: (0, 0)),
            pl.BlockSpec((D, DE), lambda i: (0, 0)),
            pl.BlockSpec((H, D), lambda i: (0, 0)),
        ],
        out_specs=[
            pl.BlockSpec((NB, D), lambda i: (i, 0)),
            pl.BlockSpec((NB, DE), lambda i: (i, 0)),
            pl.BlockSpec((NB, DE), lambda i: (i, 0)),
        ],
        out_shape=[
            jax.ShapeDtypeStruct((N, D), jnp.float32),
            jax.ShapeDtypeStruct((N, DE), jnp.float32),
            jax.ShapeDtypeStruct((N, DE), jnp.float32),
        ],
    )(accn, accd, skip, W1a, W1b, K)


B2 = 80
CHUNKS2 = EPW // B2


def _sc_edge_mlp(src, dst, Ap, Bp, ec):
    mesh = plsc.VectorSubcoreMesh(core_axis_name="c", subcore_axis_name="s")

    @functools.partial(
        pl.kernel,
        out_type=jax.ShapeDtypeStruct((E, DE), jnp.float32),
        mesh=mesh,
        compiler_params=pltpu.CompilerParams(
            use_tc_tiling_on_sc=False, needs_layout_passes=False),
        scratch_types=[
            pltpu.VMEM((B2,), jnp.int32),
            pltpu.VMEM((B2,), jnp.int32),
            pltpu.VMEM((B2, DE), jnp.float32),
            pltpu.VMEM((B2, DE), jnp.float32),
            pltpu.VMEM((B2, DE), jnp.float32),
            pltpu.VMEM((B2, DE), jnp.float32),
            pltpu.SemaphoreType.DMA,
            pltpu.SemaphoreType.DMA,
        ],
    )
    def sc2(src_hbm, dst_hbm, a_hbm, b_hbm, ec_hbm, hl_hbm,
            sidx, didx, arows, brows, ecrows, hlrows, sem_a, sem_b):
        c = lax.axis_index("c")
        s = lax.axis_index("s")
        wid = s * NC + c

        def chunk(ch, carry):
            base = wid * EPW + ch * B2
            pltpu.sync_copy(src_hbm.at[pl.ds(base, B2)], sidx)
            pltpu.sync_copy(dst_hbm.at[pl.ds(base, B2)], didx)
            cp_a = pltpu.async_copy(a_hbm.at[sidx], arows, sem_a)
            cp_b = pltpu.async_copy(b_hbm.at[didx], brows, sem_b)
            pltpu.sync_copy(ec_hbm.at[pl.ds(base, B2)], ecrows)
            cp_a.wait()
            cp_b.wait()

            def edge(e, ecarry):
                h = (arows[e, pl.ds(0, 16)] + brows[e, pl.ds(0, 16)]
                     + ecrows[e, pl.ds(0, 16)])
                hlrows[e, pl.ds(0, 16)] = jnp.maximum(h, 0.01 * h)
                return ecarry

            lax.fori_loop(0, B2, edge, 0)
            pltpu.sync_copy(hlrows, hl_hbm.at[pl.ds(base, B2)])
            return carry

        lax.fori_loop(0, CHUNKS2, chunk, 0)

    return sc2(src, dst, Ap, Bp, ec)


def _tc_edges_post(hl, edge_attr, W2, b2):
    EB = 4000

    def body(hl_ref, ea_ref, w2, b2_, out_ref):
        h2 = jnp.dot(hl_ref[...], w2[...], preferred_element_type=jnp.float32) + b2_[...]
        out_ref[...] = _leaky(ea_ref[...] + h2)

    return pl.pallas_call(
        body,
        grid=(E // EB,),
        in_specs=[
            pl.BlockSpec((EB, DE), lambda i: (i, 0)),
            pl.BlockSpec((EB, DE), lambda i: (i, 0)),
            pl.BlockSpec((DE, DE), lambda i: (0, 0)),
            pl.BlockSpec((DE,), lambda i: (0,)),
        ],
        out_specs=pl.BlockSpec((EB, DE), lambda i: (i, 0)),
        out_shape=jax.ShapeDtypeStruct((E, DE), jnp.float32),
    )(hl, edge_attr, W2, b2)


def kernel(x, edge_index, edge_attr, batch, Wq, bq, Wk, bk, Wv, bv, We,
           Wskip, bskip, W1, b1, W2, b2):
    ei = edge_index.astype(jnp.int32)
    src = ei[0]
    dst = ei[1]

    q, k, v, skip = _tc_nodes_pre(x, Wq, bq, Wk, bk, Wv, bv, Wskip, bskip)
    ea, ec = _tc_edges_pre(edge_attr, We, W1[2 * D:], b1)
    accn, accd = _sc_attention(ei, q, k, v, ea)

    K = jnp.asarray(np.kron(np.eye(H), np.ones((1, C))), dtype=jnp.float32)
    x_new, Ap, Bp = _tc_nodes_post(accn, accd, skip, W1[:D], W1[D:2 * D], K)

    hl = _sc_edge_mlp(src, dst, Ap, Bp, ec)
    edge_new = _tc_edges_post(hl, edge_attr, W2, b2)
    return (x_new, edge_new)


# speedup vs baseline: 24.9000x; 1.0968x over previous
import functools

import jax
import jax.numpy as jnp
import numpy as np
from jax import lax
from jax.experimental import pallas as pl
from jax.experimental.pallas import tpu as pltpu
from jax.experimental.pallas import tpu_sc as plsc

N = 10000
E = 320000
D = 128
DE = 16
H = 16
C = 8

NC = 2
NS = 16
NW = NC * NS
EPW = E // NW
B = 32
CHUNKS = EPW // B
TB = EPW - CHUNKS * B
RPT = N // NS
AW = D + H

_INV_SQRT_C = 1.0 / float(np.sqrt(C))


def _leaky(v):
    return jnp.maximum(v, 0.01 * v)


_GD = lax.GatherDimensionNumbers(offset_dims=(), collapsed_slice_dims=(0,),
                                 start_index_map=(0,))


def _vtake(x, idx):
    return lax.gather(x, idx[:, None], _GD, (1,),
                      mode=lax.GatherScatterMode.PROMISE_IN_BOUNDS)


def _tc_nodes_pre(x, Wq, bq, Wk, bk, Wv, bv, Wskip, bskip):
    NB = 2000

    def body(x_ref, wq, bq_, wk, bk_, wv, bv_, ws, bs_, q_ref, k_ref, v_ref, s_ref):
        xb = x_ref[...]
        q_ref[...] = jnp.dot(xb, wq[...], preferred_element_type=jnp.float32) + bq_[...]
        k_ref[...] = jnp.dot(xb, wk[...], preferred_element_type=jnp.float32) + bk_[...]
        v_ref[...] = jnp.dot(xb, wv[...], preferred_element_type=jnp.float32) + bv_[...]
        s_ref[...] = jnp.dot(xb, ws[...], preferred_element_type=jnp.float32) + bs_[...]

    wspec = pl.BlockSpec((D, D), lambda i: (0, 0))
    bspec = pl.BlockSpec((D,), lambda i: (0,))
    nspec = pl.BlockSpec((NB, D), lambda i: (i, 0))
    return pl.pallas_call(
        body,
        grid=(N // NB,),
        in_specs=[
            nspec,
            wspec, bspec, wspec, bspec, wspec, bspec, wspec, bspec,
        ],
        out_specs=[nspec, nspec, nspec, nspec],
        out_shape=[
            jax.ShapeDtypeStruct((N, D), jnp.float32),
            jax.ShapeDtypeStruct((N, D), jnp.float32),
            jax.ShapeDtypeStruct((N, D), jnp.float32),
            jax.ShapeDtypeStruct((N, D), jnp.float32),
        ],
    )(x, Wq, bq, Wk, bk, Wv, bv, Wskip, bskip)


def _tc_edges_pre(edge_attr, We, W1c, b1):
    EB = 4000

    def body(ea_ref, we, w1c, b1_, eaw_ref, ec_ref):
        eb = ea_ref[...]
        eaw_ref[...] = jnp.dot(eb, we[...], preferred_element_type=jnp.float32)
        ec_ref[...] = jnp.dot(eb, w1c[...], preferred_element_type=jnp.float32) + b1_[...]

    return pl.pallas_call(
        body,
        grid=(E // EB,),
        in_specs=[
            pl.BlockSpec((EB, DE), lambda i: (i, 0)),
            pl.BlockSpec((DE, D), lambda i: (0, 0)),
            pl.BlockSpec((DE, DE), lambda i: (0, 0)),
            pl.BlockSpec((DE,), lambda i: (0,)),
        ],
        out_specs=[
            pl.BlockSpec((EB, D), lambda i: (i, 0)),
            pl.BlockSpec((EB, DE), lambda i: (i, 0)),
        ],
        out_shape=[
            jax.ShapeDtypeStruct((E, D), jnp.float32),
            jax.ShapeDtypeStruct((E, DE), jnp.float32),
        ],
    )(edge_attr, We, W1c, b1)


def _sc_attention(ei, q, k, v, ea):
    mesh = plsc.VectorSubcoreMesh(core_axis_name="c", subcore_axis_name="s")

    @functools.partial(
        pl.kernel,
        out_type=[
            jax.ShapeDtypeStruct((NC, N, D), jnp.float32),
            jax.ShapeDtypeStruct((NC, N, H), jnp.float32),
        ],
        mesh=mesh,
        compiler_params=pltpu.CompilerParams(
            use_tc_tiling_on_sc=False, needs_layout_passes=False),
        scratch_types=[
            pltpu.VMEM_SHARED((N, D), jnp.float32),
            pltpu.VMEM_SHARED((N, H), jnp.float32),
            pltpu.VMEM((2, B), jnp.int32),
            pltpu.VMEM((2, B), jnp.int32),
            pltpu.VMEM((B,), jnp.int32),
            pltpu.VMEM((B,), jnp.int32),
            pltpu.VMEM((B, D), jnp.float32),
            pltpu.VMEM((B, D), jnp.float32),
            pltpu.VMEM((B, D), jnp.float32),
            pltpu.VMEM((B, D), jnp.float32),
            pltpu.VMEM((B, D), jnp.float32),
            pltpu.VMEM((B, D), jnp.float32),
            pltpu.VMEM((B, H), jnp.float32),
            pltpu.VMEM((B, H), jnp.float32),
            pltpu.VMEM((B, D), jnp.float32),
            pltpu.SemaphoreType.DMA,
            pltpu.SemaphoreType.DMA,
            pltpu.SemaphoreType.DMA,
            pltpu.SemaphoreType.DMA,
            pltpu.SemaphoreType.DMA,
            pltpu.SemaphoreType.DMA,
        ],
    )
    def sc1(ei_hbm, q_hbm, k_hbm, v_hbm, ea_hbm, outn_hbm, outd_hbm,
            accN, accD, eib0, eib1, didx0, didx1,
            q0, q1, k0, k1, v0, v1, ax0, ax1, earows,
            sem_q, sem_k, sem_v, sem_ea, sem_sc0, sem_sc1):
        c = lax.axis_index("c")
        s = lax.axis_index("s")
        wid = s * NC + c

        zero16 = jnp.zeros((16,), jnp.float32)

        def zb(i, carry):
            for j in range(D // 16):
                k0[i, pl.ds(16 * j, 16)] = zero16
            ax0[i, pl.ds(0, 16)] = zero16
            return carry

        lax.fori_loop(0, B, zb, 0)
        nfull = RPT // B
        for kblk in range(nfull):
            r0 = s * RPT + kblk * B
            pltpu.sync_copy(k0, accN.at[pl.ds(r0, B)])
            pltpu.sync_copy(ax0, accD.at[pl.ds(r0, B)])
        rem = RPT - nfull * B
        if rem:
            r0 = s * RPT + nfull * B
            pltpu.sync_copy(k0.at[pl.ds(0, rem)], accN.at[pl.ds(r0, rem)])
            pltpu.sync_copy(ax0.at[pl.ds(0, rem)], accD.at[pl.ds(r0, rem)])
        plsc.subcore_barrier()

        iota = lax.iota(jnp.int32, 16)
        sumidx = [iota * 8 + cc for cc in range(8)]
        bsel = jnp.where(iota >= 8, 1, 0).astype(jnp.int32)

        slots = [(eib0, didx0, q0, k0, v0, ax0, sem_sc0),
                 (eib1, didx1, q1, k1, v1, ax1, sem_sc1)]

        def base_of(ch):
            return wid * EPW + ch * B

        def issue_g(eib_x, q_slot, k_slot, v_slot):
            pltpu.async_copy(q_hbm.at[eib_x.at[1]], q_slot, sem_q)
            pltpu.async_copy(k_hbm.at[eib_x.at[0]], k_slot, sem_k)
            pltpu.async_copy(v_hbm.at[eib_x.at[0]], v_slot, sem_v)

        def wait_g(eib_x, q_slot, k_slot, v_slot):
            pltpu.make_async_copy(q_hbm.at[eib_x.at[1]], q_slot, sem_q).wait()
            pltpu.make_async_copy(k_hbm.at[eib_x.at[0]], k_slot, sem_k).wait()
            pltpu.make_async_copy(v_hbm.at[eib_x.at[0]], v_slot, sem_v).wait()

        def issue_ea(base):
            pltpu.async_copy(ea_hbm.at[pl.ds(base, B)], earows, sem_ea)

        def wait_ea(base):
            pltpu.make_async_copy(ea_hbm.at[pl.ds(base, B)], earows, sem_ea).wait()

        def issue_sc(k_slot, ax_slot, didx_x, sem):
            pltpu.async_copy(k_slot, accN.at[didx_x], sem, add=True)
            pltpu.async_copy(ax_slot, accD.at[didx_x], sem, add=True)

        def drain_sc(k_slot, ax_slot, didx_x, sem):
            pltpu.make_async_copy(k_slot, accN.at[didx_x], sem).wait()
            pltpu.make_async_copy(ax_slot, accD.at[didx_x], sem).wait()

        def compute_chunk(qr, kr, vr, axr):
            @plsc.parallel_loop(0, B, unroll=4)
            def edge(e):
                erow = jnp.full((16,), e, dtype=jnp.int32)
                eav = []
                vv = []
                for j in range(8):
                    ea_j = earows[e, pl.ds(16 * j, 16)]
                    eav.append(ea_j)
                    vv.append(vr[e, pl.ds(16 * j, 16)])
                    kj = kr[e, pl.ds(16 * j, 16)] + ea_j
                    kr[e, pl.ds(16 * j, 16)] = qr[e, pl.ds(16 * j, 16)] * kj
                al = plsc.load_gather(kr, [erow, sumidx[0]])
                for cc in range(1, 8):
                    al = al + plsc.load_gather(kr, [erow, sumidx[cc]])
                aexp = jnp.exp(al * _INV_SQRT_C)
                axr[e, pl.ds(0, 16)] = aexp
                for j in range(8):
                    bc = _vtake(aexp, bsel + 2 * j)
                    kr[e, pl.ds(16 * j, 16)] = (vv[j] + eav[j]) * bc

        def body(ch, b, first, last):
            eib_b, didx_b, qb, kb, vb, axb, scb = slots[b]
            eib_n, didx_n, qn, kn, vn, axn, scn = slots[1 - b]
            wait_ea(base_of(ch))
            wait_g(eib_b, qb, kb, vb)
            compute_chunk(qb, kb, vb, axb)
            issue_sc(kb, axb, didx_b, scb)
            if not first:
                drain_sc(kn, axn, didx_n, scn)
            if not last:
                nb = base_of(ch + 1)
                pltpu.sync_copy(ei_hbm.at[:, pl.ds(nb, B)], eib_n)
                pltpu.sync_copy(ei_hbm.at[1, pl.ds(nb, B)], didx_n)
                issue_g(eib_n, qn, kn, vn)
                issue_ea(nb)

        base0 = base_of(0)
        pltpu.sync_copy(ei_hbm.at[:, pl.ds(base0, B)], eib0)
        pltpu.sync_copy(ei_hbm.at[1, pl.ds(base0, B)], didx0)
        issue_g(eib0, q0, k0, v0)
        issue_ea(base0)
        body(0, 0, first=True, last=False)

        def pair(p, carry):
            ch = 1 + 2 * p
            body(ch, 1, first=False, last=False)
            body(ch + 1, 0, first=False, last=False)
            return carry

        lax.fori_loop(0, (CHUNKS - 2) // 2, pair, 0)
        body(CHUNKS - 1, 1, first=False, last=True)
        drain_sc(k1, ax1, didx1, sem_sc1)

        base_t = wid * EPW + EPW - B
        pltpu.sync_copy(ei_hbm.at[:, pl.ds(base_t, B)], eib0)
        pltpu.sync_copy(ei_hbm.at[1, pl.ds(base_t, B)], didx0)
        issue_g(eib0, q0, k0, v0)
        issue_ea(base_t)
        wait_ea(base_t)
        wait_g(eib0, q0, k0, v0)
        compute_chunk(q0, k0, v0, ax0)
        zero16f = jnp.zeros((16,), jnp.float32)

        def ztail(e, carry):
            for j in range(D // 16):
                k0[e, pl.ds(16 * j, 16)] = zero16f
            ax0[e, pl.ds(0, 16)] = zero16f
            return carry

        lax.fori_loop(0, B - TB, ztail, 0)
        pltpu.sync_copy(k0, accN.at[didx0], add=True)
        pltpu.sync_copy(ax0, accD.at[didx0], add=True)
        plsc.subcore_barrier()
        pltpu.sync_copy(accN.at[pl.ds(s * RPT, RPT)],
                        outn_hbm.at[c, pl.ds(s * RPT, RPT)])
        pltpu.sync_copy(accD.at[pl.ds(s * RPT, RPT)],
                        outd_hbm.at[c, pl.ds(s * RPT, RPT)])

    return sc1(ei, q, k, v, ea)


def _tc_nodes_post(accn, accd, skip, W1a, W1b, K):
    NB = 2000

    def body(an_ref, ad_ref, s_ref, w1a, w1b, k_ref, xn_ref, a_ref, b_ref):
        an = an_ref[...]
        ad = ad_ref[...]
        num = an[0] + an[1]
        den = ad[0] + ad[1]
        inv = 1.0 / (den + 1e-16)
        ratio = num * jnp.dot(inv, k_ref[...], preferred_element_type=jnp.float32)
        out = ratio + s_ref[...]
        xn = _leaky(out)
        xn_ref[...] = xn
        a_ref[...] = jnp.dot(xn, w1a[...], preferred_element_type=jnp.float32)
        b_ref[...] = jnp.dot(xn, w1b[...], preferred_element_type=jnp.float32)

    return pl.pallas_call(
        body,
        grid=(N // NB,),
        in_specs=[
            pl.BlockSpec((NC, NB, D), lambda i: (0, i, 0)),
            pl.BlockSpec((NC, NB, H), lambda i: (0, i, 0)),
            pl.BlockSpec((NB, D), lambda i: (i, 0)),
            pl.BlockSpec((D, DE), lambda i: (0, 0)),
            pl.BlockSpec((D, DE), lambda i: (0, 0)),
            pl.BlockSpec((H, D), lambda i: (0, 0)),
        ],
        out_specs=[
            pl.BlockSpec((NB, D), lambda i: (i, 0)),
            pl.BlockSpec((NB, DE), lambda i: (i, 0)),
            pl.BlockSpec((NB, DE), lambda i: (i, 0)),
        ],
        out_shape=[
            jax.ShapeDtypeStruct((N, D), jnp.float32),
            jax.ShapeDtypeStruct((N, DE), jnp.float32),
            jax.ShapeDtypeStruct((N, DE), jnp.float32),
        ],
    )(accn, accd, skip, W1a, W1b, K)


B2 = 80
CHUNKS2 = EPW // B2


def _sc_edge_mlp(src, dst, Ap, Bp, ec):
    mesh = plsc.VectorSubcoreMesh(core_axis_name="c", subcore_axis_name="s")

    @functools.partial(
        pl.kernel,
        out_type=jax.ShapeDtypeStruct((E, DE), jnp.float32),
        mesh=mesh,
        compiler_params=pltpu.CompilerParams(
            use_tc_tiling_on_sc=False, needs_layout_passes=False),
        scratch_types=[
            pltpu.VMEM((B2,), jnp.int32),
            pltpu.VMEM((B2,), jnp.int32),
            pltpu.VMEM((B2, DE), jnp.float32),
            pltpu.VMEM((B2, DE), jnp.float32),
            pltpu.VMEM((B2, DE), jnp.float32),
            pltpu.VMEM((B2, DE), jnp.float32),
            pltpu.SemaphoreType.DMA,
            pltpu.SemaphoreType.DMA,
        ],
    )
    def sc2(src_hbm, dst_hbm, a_hbm, b_hbm, ec_hbm, hl_hbm,
            sidx, didx, arows, brows, ecrows, hlrows, sem_a, sem_b):
        c = lax.axis_index("c")
        s = lax.axis_index("s")
        wid = s * NC + c

        def chunk(ch, carry):
            base = wid * EPW + ch * B2
            pltpu.sync_copy(src_hbm.at[pl.ds(base, B2)], sidx)
            pltpu.sync_copy(dst_hbm.at[pl.ds(base, B2)], didx)
            cp_a = pltpu.async_copy(a_hbm.at[sidx], arows, sem_a)
            cp_b = pltpu.async_copy(b_hbm.at[didx], brows, sem_b)
            pltpu.sync_copy(ec_hbm.at[pl.ds(base, B2)], ecrows)
            cp_a.wait()
            cp_b.wait()

            @plsc.parallel_loop(0, B2, unroll=8)
            def edge(e):
                h = (arows[e, pl.ds(0, 16)] + brows[e, pl.ds(0, 16)]
                     + ecrows[e, pl.ds(0, 16)])
                hlrows[e, pl.ds(0, 16)] = jnp.maximum(h, 0.01 * h)
            pltpu.sync_copy(hlrows, hl_hbm.at[pl.ds(base, B2)])
            return carry

        lax.fori_loop(0, CHUNKS2, chunk, 0)

    return sc2(src, dst, Ap, Bp, ec)


def _tc_edges_post(hl, edge_attr, W2, b2):
    EB = 4000

    def body(hl_ref, ea_ref, w2, b2_, out_ref):
        h2 = jnp.dot(hl_ref[...], w2[...], preferred_element_type=jnp.float32) + b2_[...]
        out_ref[...] = _leaky(ea_ref[...] + h2)

    return pl.pallas_call(
        body,
        grid=(E // EB,),
        in_specs=[
            pl.BlockSpec((EB, DE), lambda i: (i, 0)),
            pl.BlockSpec((EB, DE), lambda i: (i, 0)),
            pl.BlockSpec((DE, DE), lambda i: (0, 0)),
            pl.BlockSpec((DE,), lambda i: (0,)),
        ],
        out_specs=pl.BlockSpec((EB, DE), lambda i: (i, 0)),
        out_shape=jax.ShapeDtypeStruct((E, DE), jnp.float32),
    )(hl, edge_attr, W2, b2)


def kernel(x, edge_index, edge_attr, batch, Wq, bq, Wk, bk, Wv, bv, We,
           Wskip, bskip, W1, b1, W2, b2):
    ei = edge_index.astype(jnp.int32)
    src = ei[0]
    dst = ei[1]

    q, k, v, skip = _tc_nodes_pre(x, Wq, bq, Wk, bk, Wv, bv, Wskip, bskip)
    ea, ec = _tc_edges_pre(edge_attr, We, W1[2 * D:], b1)
    accn, accd = _sc_attention(ei, q, k, v, ea)

    K = jnp.asarray(np.kron(np.eye(H), np.ones((1, C))), dtype=jnp.float32)
    x_new, Ap, Bp = _tc_nodes_post(accn, accd, skip, W1[:D], W1[D:2 * D], K)

    hl = _sc_edge_mlp(src, dst, Ap, Bp, ec)
    edge_new = _tc_edges_post(hl, edge_attr, W2, b2)
    return (x_new, edge_new)
